# baseline (speedup 1.0000x reference)
import functools

import jax
import jax.numpy as jnp
from jax import lax
from jax.experimental import pallas as pl
from jax.experimental.pallas import tpu as pltpu
from jax.experimental.pallas import tpu_sc as plsc

B, N, D = 1024, 200, 128
_SCALE = float(D) ** 0.5

_NW = 32
_BPW = B // _NW
_NB = 8
_NI = 8
_NBG = _BPW // _NB
_NIG = N // _NI
_T = _NBG * _NIG
_MA = _BPW * N


def _sc_body(len_hbm, ids_hbm, r_hbm, pe_hbm, posm_hbm, t7_hbm,
             l2_hbm, ue_hbm, me_hbm, mo_hbm,
             posm_s, t7_v, idsb_v, rb_v, pe_s, out_s,
             me_all, mo_all, r_all, len_v, in_sem, out_sem):
    c = lax.axis_index("c")
    s = lax.axis_index("s")
    w = s * 2 + c
    b0 = pl.multiple_of(w * _BPW, 8)

    pltpu.sync_copy(t7_hbm, t7_v)
    pltpu.sync_copy(ids_hbm.at[pl.ds(b0, _BPW)], idsb_v)
    pltpu.sync_copy(r_hbm.at[pl.ds(b0, _BPW)], rb_v)

    pltpu.sync_copy(len_hbm.at[pl.ds(b0, _BPW)], len_v)
    for g in range(_BPW // 16):
        len_v[pl.ds(g * 16, 16)] = len_v[pl.ds(g * 16, 16)] * 2
    pltpu.sync_copy(len_v, l2_hbm.at[pl.ds(b0, _BPW)])

    def stage_row(bl, _):
        base = bl * N
        for goff in list(range(0, N - 16, 16)) + [N - 16]:
            ids16 = idsb_v[bl, pl.ds(goff, 16)]
            r16 = rb_v[bl, pl.ds(goff, 16)]
            me_all[pl.ds(base + goff, 16)] = jnp.where(
                ids16 != 0, 1.0, 0.0).astype(jnp.float32)
            mo_all[pl.ds(base + goff, 16)] = jnp.where(
                (r16 != 0) & (r16 != 6), 1.0, 0.0).astype(jnp.float32)
            r_all[pl.ds(base + goff, 16)] = r16
        return _

    lax.fori_loop(0, _BPW, stage_row, 0)

    def in_copies(t, slot):
        bsub = pl.multiple_of(b0 + (t // _NIG) * _NB, 8)
        i0 = pl.multiple_of((t % _NIG) * _NI, 8)
        return (
            pltpu.make_async_copy(pe_hbm.at[pl.ds(bsub, _NB), pl.ds(i0, _NI)],
                                  pe_s.at[slot], in_sem.at[slot]),
            pltpu.make_async_copy(posm_hbm.at[pl.ds(i0, _NI)],
                                  posm_s.at[slot], in_sem.at[slot]),
        )

    def out_copies(t, slot):
        bsub = pl.multiple_of(b0 + (t // _NIG) * _NB, 8)
        i0 = pl.multiple_of((t % _NIG) * _NI, 8)
        return (
            pltpu.make_async_copy(
                out_s.at[slot],
                ue_hbm.at[pl.ds(bsub, _NB), pl.ds(2 * i0, 2 * _NI)],
                out_sem.at[slot]),
        )

    def compute(t, slot):
        bg = t // _NIG
        i0 = (t % _NIG) * _NI
        mbase = bg * _NB * N + i0
        for ii in range(_NI):
            pme = [posm_s[slot, ii, pl.ds(k * 16, 16)] for k in range(8)]
            pmo = [posm_s[slot, ii, pl.ds(D + k * 16, 16)] for k in range(8)]

            @plsc.parallel_loop(0, _NB, 1, unroll=2)
            def pair(bb):
                mo_off = mbase + bb * N + ii
                me_s = lax.broadcast(me_all[pl.ds(mo_off, 16)][0], (16,))
                mo_s = lax.broadcast(mo_all[pl.ds(mo_off, 16)][0], (16,))
                r = r_all[pl.ds(mo_off, 16)][0]
                for k in range(8):
                    ev = pe_s[slot, bb, ii, pl.ds(k * 16, 16)] * _SCALE
                    ev = (ev + pme[k]) * me_s
                    out_s[slot, bb, 2 * ii, pl.ds(k * 16, 16)] = ev
                    ov = (t7_v[r, pl.ds(k * 16, 16)] + pmo[k]) * mo_s
                    out_s[slot, bb, 2 * ii + 1, pl.ds(k * 16, 16)] = ov

    for dma in in_copies(0, 0):
        dma.start()

    def step(t, _):
        slot = lax.rem(t, 2)

        @pl.when(t + 1 < _T)
        def _prefetch():
            for dma in in_copies(t + 1, 1 - slot):
                dma.start()

        for dma in in_copies(t, slot):
            dma.wait()

        @pl.when(t >= 2)
        def _drain_out():
            for dma in out_copies(t - 2, slot):
                dma.wait()

        compute(t, slot)
        for dma in out_copies(t, slot):
            dma.start()
        return _

    lax.fori_loop(0, _T, step, 0)
    for dma in out_copies(_T - 2, 0):
        dma.wait()
    for dma in out_copies(_T - 1, 1):
        dma.wait()

    fo = pl.multiple_of(b0 * N, 8)
    pltpu.sync_copy(me_all.at[pl.ds(0, _MA)], me_hbm.at[pl.ds(fo, _MA)])
    pltpu.sync_copy(mo_all.at[pl.ds(0, _MA)], mo_hbm.at[pl.ds(fo, _MA)])


@jax.jit
def _run_sc(past_lengths, past_ids, ratings, past_embeddings, posm, t7):
    mesh = plsc.VectorSubcoreMesh(core_axis_name="c", subcore_axis_name="s")
    f = functools.partial(
        pl.kernel,
        mesh=mesh,
        out_type=[
            jax.ShapeDtypeStruct((B,), jnp.int32),
            jax.ShapeDtypeStruct((B, 2 * N, D), jnp.float32),
            jax.ShapeDtypeStruct((B * N,), jnp.float32),
            jax.ShapeDtypeStruct((B * N,), jnp.float32),
        ],
        scratch_types=[
            pltpu.VMEM((2, _NI, 2 * D), jnp.float32),
            pltpu.VMEM((8, D), jnp.float32),
            pltpu.VMEM((_BPW, N), jnp.int32),
            pltpu.VMEM((_BPW, N), jnp.int32),
            pltpu.VMEM((2, _NB, _NI, D), jnp.float32),
            pltpu.VMEM((2, _NB, 2 * _NI, D), jnp.float32),
            pltpu.VMEM((_MA + 16,), jnp.float32),
            pltpu.VMEM((_MA + 16,), jnp.float32),
            pltpu.VMEM((_MA + 16,), jnp.int32),
            pltpu.VMEM((_BPW,), jnp.int32),
            pltpu.SemaphoreType.DMA((2,)),
            pltpu.SemaphoreType.DMA((2,)),
        ],
    )(_sc_body)
    return f(past_lengths, past_ids, ratings, past_embeddings, posm, t7)


def kernel(past_lengths, past_ids, past_embeddings, ratings, pos_emb,
           iasig_emb, rating_emb):
    posm = (pos_emb + iasig_emb[jnp.arange(2 * N) % 2]).reshape(N, 2 * D)
    t7 = jnp.concatenate([rating_emb * _SCALE,
                          jnp.zeros((1, D), jnp.float32)], axis=0)
    l2, ue, me, mo = _run_sc(past_lengths, past_ids, ratings,
                             past_embeddings, posm, t7)
    m = jnp.stack([me.reshape(B, N), mo.reshape(B, N)], axis=-1)
    return (l2, ue, m.reshape(B, 2 * N, 1))
